# Initial kernel scaffold; baseline (speedup 1.0000x reference)
import jax
import jax.numpy as jnp
from jax.experimental import pallas as pl


def kernel(position_ids, table):
    raise NotImplementedError("write your pallas kernel here")


# speedup vs baseline: 2.1738x; 2.1738x over previous
import functools

import jax
import jax.numpy as jnp
from jax import lax
from jax.experimental import pallas as pl
from jax.experimental.pallas import tpu as pltpu
from jax.experimental.pallas import tpu_sc as plsc

_B = 32768
_D = 1024
_NC = 2
_NS = 16
_NW = _NC * _NS
_BPW = _B // _NW
_CH = 64
_NCH = _BPW // _CH


@jax.jit
def _sc_gather(idx, table):
  mesh = plsc.VectorSubcoreMesh(core_axis_name="c", subcore_axis_name="s")

  @functools.partial(
      pl.kernel,
      out_type=jax.ShapeDtypeStruct((_B, _D), jnp.float32),
      mesh=mesh,
      scratch_types=[
          pltpu.VMEM((_NCH, _CH), jnp.int32),
          pltpu.VMEM((_CH, _D), jnp.float32),
          pltpu.SemaphoreType.DMA,
      ],
  )
  def k(idx_hbm, table_hbm, out_hbm, idx_v, rows_v, sem):
    wid = lax.axis_index("s") * _NC + lax.axis_index("c")
    base = wid * _BPW
    pltpu.sync_copy(idx_hbm.at[wid], idx_v)

    def body(c, carry):
      pltpu.async_copy(table_hbm.at[idx_v.at[c]], rows_v, sem).wait()
      pltpu.sync_copy(rows_v, out_hbm.at[pl.ds(base + c * _CH, _CH)])
      return carry

    lax.fori_loop(0, _NCH, body, 0)

  return k(idx, table)


def kernel(position_ids, table):
  idx = position_ids.reshape(_NW, _NCH, _CH).astype(jnp.int32)
  out = _sc_gather(idx, table)
  return out.reshape(position_ids.shape + (table.shape[1],))


# speedup vs baseline: 2.3634x; 1.0872x over previous
import functools

import jax
import jax.numpy as jnp
from jax import lax
from jax.experimental import pallas as pl
from jax.experimental.pallas import tpu as pltpu
from jax.experimental.pallas import tpu_sc as plsc

_B = 32768
_D = 1024
_NC = 2
_NS = 16
_NW = _NC * _NS
_BPW = _B // _NW
_CH = 32
_NCH = _BPW // _CH


@jax.jit
def _sc_gather(idx, table):
  mesh = plsc.VectorSubcoreMesh(core_axis_name="c", subcore_axis_name="s")

  @functools.partial(
      pl.kernel,
      out_type=jax.ShapeDtypeStruct((_B, _D), jnp.float32),
      mesh=mesh,
      scratch_types=[
          pltpu.VMEM((_NCH, _CH), jnp.int32),
          pltpu.VMEM((_CH, _D), jnp.float32),
          pltpu.VMEM((_CH, _D), jnp.float32),
          pltpu.SemaphoreType.DMA,
          pltpu.SemaphoreType.DMA,
      ],
  )
  def k(idx_hbm, table_hbm, out_hbm, idx_v, rows0, rows1, sem0, sem1):
    wid = lax.axis_index("s") * _NC + lax.axis_index("c")
    base = wid * _BPW
    pltpu.sync_copy(idx_hbm.at[wid], idx_v)

    pltpu.async_copy(table_hbm.at[idx_v.at[0]], rows0, sem0)

    def body(g, carry):
      c = 2 * g
      pltpu.async_copy(table_hbm.at[idx_v.at[c + 1]], rows1, sem1)
      pltpu.make_async_copy(table_hbm.at[idx_v.at[c]], rows0, sem0).wait()
      pltpu.sync_copy(rows0, out_hbm.at[pl.ds(base + c * _CH, _CH)])

      @pl.when(c + 2 < _NCH)
      def _():
        pltpu.async_copy(table_hbm.at[idx_v.at[c + 2]], rows0, sem0)

      pltpu.make_async_copy(table_hbm.at[idx_v.at[c + 1]], rows1, sem1).wait()
      pltpu.sync_copy(rows1, out_hbm.at[pl.ds(base + (c + 1) * _CH, _CH)])
      return carry

    lax.fori_loop(0, _NCH // 2, body, 0)

  return k(idx, table)


def kernel(position_ids, table):
  idx = position_ids.reshape(_NW, _NCH, _CH).astype(jnp.int32)
  out = _sc_gather(idx, table)
  return out.reshape(position_ids.shape + (table.shape[1],))
